# baseline (speedup 1.0000x reference)
import jax
import jax.numpy as jnp
import numpy as np
from jax.experimental import pallas as pl

N = 8192
N1 = 64
N2 = 128
KL1 = 72
GA = 65
GB = 64
KHA = 33
KHB = 32
F_MAX = N // 2
TOPK = 4
DT = 128


def _dft_consts():
    kl = np.arange(KL1)[:, None]
    n2 = np.arange(N2)[None, :]
    ph = -2.0 * np.pi * ((kl * n2) % N2) / N2
    d1r = np.cos(ph).astype(np.float32)
    d1i = np.sin(ph).astype(np.float32)
    d1r[GA:] = 0.0
    d1i[GA:] = 0.0

    n1 = np.arange(N1)[None, :]
    pht = -2.0 * np.pi * ((np.arange(KL1)[:, None] * n1) % N) / N
    tr = np.cos(pht).astype(np.float32)
    ti = np.sin(pht).astype(np.float32)
    tr[GA:] = 0.0
    ti[GA:] = 0.0

    kh = np.arange(KHA)[:, None]
    phe = -2.0 * np.pi * ((kh * n1) % N1) / N1
    ear = np.cos(phe).astype(np.float32)
    eai = np.sin(phe).astype(np.float32)
    eas = (ear + eai).astype(np.float32)

    khb = np.arange(KHB)[:, None] + 1
    phb = -2.0 * np.pi * ((khb * n1) % N1) / N1
    ebr = np.cos(phb).astype(np.float32)
    ebi = np.sin(phb).astype(np.float32)
    ebs = (ebr + ebi).astype(np.float32)

    def mstack(a, b):
        m = np.zeros((KL1, a.shape[1]), np.float32)
        m[:KHA] = a
        m[KHB + 8:] = b
        return m

    e1 = mstack(ear, ebr)
    e2 = mstack(eai, ebi)
    e3 = np.zeros((KL1, 2 * N1), np.float32)
    e3[:KHA, :N1] = eas
    e3[KHB + 8:, N1:] = ebs
    return d1r, d1i, tr, ti, e1, e2, e3


_CONSTS = _dft_consts()


def _fold(v, reducer, groups):
    return reducer(v.reshape(groups, DT), axis=0, keepdims=True)


def _fa_kernel(x_ref, d1r_ref, d1i_ref, tr_ref, ti_ref, e1_ref, e2_ref,
               e3_ref, out_ref):
    xb = x_ref[0]
    a2f = xb.reshape(N2, N1 * DT)
    hi_p = jax.lax.Precision.HIGHEST
    gr = jnp.dot(d1r_ref[...], a2f, precision=hi_p)
    gi = jnp.dot(d1i_ref[...], a2f, precision=hi_p)
    gr3 = gr.reshape(KL1, N1, DT)
    gi3 = gi.reshape(KL1, N1, DT)
    tr3 = tr_ref[...][:, :, None]
    ti3 = ti_ref[...][:, :, None]
    hr = gr3 * tr3 - gi3 * ti3
    hi = gr3 * ti3 + gi3 * tr3
    hpr = jnp.transpose(hr[:GA], (1, 0, 2)).reshape(N1, GA * DT)
    hpi = jnp.transpose(hi[:GA], (1, 0, 2)).reshape(N1, GA * DT)
    hps = hpr + hpi
    hpd = hpr - hpi
    rhs3 = jnp.concatenate([hps, hpd], axis=0)
    r1 = jnp.dot(e1_ref[...], hpr, precision=hi_p)
    r2 = jnp.dot(e2_ref[...], hpi, precision=hi_p)
    r3 = jnp.dot(e3_ref[...], rhs3, precision=hi_p)
    t1 = r1[:KHA]
    t2 = r2[:KHA]
    t3 = r3[:KHA]
    u1 = r1[KHB + 8:]
    u2 = r2[KHB + 8:]
    u3 = r3[KHB + 8:]
    x2ar = t1 - t2
    x2ai = t3 - t1 - t2
    x2br = u1 + u2
    x2bi = u3 - u1 + u2

    big_i = jnp.int32(2**30)
    kh_a = jax.lax.broadcasted_iota(jnp.int32, (KHA, GA * DT), 0)
    ln_a = jax.lax.broadcasted_iota(jnp.int32, (KHA, GA * DT), 1)
    kidx_a = kh_a * N2 + jax.lax.shift_right_logical(ln_a, 7)
    amp_a = x2ar * x2ar + x2ai * x2ai
    amp_a = jnp.where(kidx_a <= F_MAX, amp_a, -1.0)

    kh_b = jax.lax.broadcasted_iota(jnp.int32, (KHB, GA * DT), 0)
    ln_b = jax.lax.broadcasted_iota(jnp.int32, (KHB, GA * DT), 1)
    grp_b = jax.lax.shift_right_logical(ln_b, 7)
    dup_b = (grp_b == 0) | (grp_b == N1)
    kidx_b = jnp.where(dup_b, big_i, (kh_b + 1) * N2 - grp_b)
    amp_b = jnp.where(dup_b, -1.0, x2br * x2br + x2bi * x2bi)

    out3 = jnp.zeros((N2, N1, DT), jnp.bfloat16)
    a_io = jax.lax.broadcasted_iota(jnp.int32, (N1, DT), 0)
    b_io = jax.lax.broadcasted_iota(jnp.int32, (N2, DT), 0)
    wka = amp_a
    wkb = amp_b
    inv_n = np.float32(1.0 / N)
    for _ in range(TOPK):
        m = jnp.maximum(_fold(jnp.max(wka, axis=0, keepdims=True), jnp.max, GA),
                        _fold(jnp.max(wkb, axis=0, keepdims=True), jnp.max, GA))
        m_a = jnp.tile(m, (1, GA))
        m_b = jnp.tile(m, (1, GA))
        cand_a = jnp.where(wka == m_a, kidx_a, big_i)
        cand_b = jnp.where(wkb == m_b, kidx_b, big_i)
        f_sel = jnp.minimum(
            _fold(jnp.min(cand_a, axis=0, keepdims=True), jnp.min, GA),
            _fold(jnp.min(cand_b, axis=0, keepdims=True), jnp.min, GA))
        f_a = jnp.tile(f_sel, (1, GA))
        f_b = jnp.tile(f_sel, (1, GA))
        oh_a = kidx_a == f_a
        oh_b = kidx_b == f_b
        wr = (_fold(jnp.sum(jnp.where(oh_a, x2ar, 0.0), axis=0, keepdims=True),
                    jnp.sum, GA)
              + _fold(jnp.sum(jnp.where(oh_b, x2br, 0.0), axis=0, keepdims=True),
                      jnp.sum, GA))
        wi = (_fold(jnp.sum(jnp.where(oh_a, x2ai, 0.0), axis=0, keepdims=True),
                    jnp.sum, GA)
              + _fold(jnp.sum(jnp.where(oh_b, x2bi, 0.0), axis=0, keepdims=True),
                      jnp.sum, GA))
        wka = jnp.where(oh_a, -2.0, wka)
        wkb = jnp.where(oh_b, -2.0, wkb)
        scale = jnp.where((f_sel == 0) | (f_sel == F_MAX), inv_n,
                          np.float32(2.0 / N))
        cr = wr * scale
        ci = wi * scale
        fa = (a_io * f_sel) & (N - 1)
        th_a = fa.astype(jnp.float32) * np.float32(2.0 * np.pi / N)
        uc = jnp.cos(th_a)
        us = jnp.sin(th_a)
        fb = (b_io * f_sel) & (N2 - 1)
        th_b = fb.astype(jnp.float32) * np.float32(2.0 * np.pi / N2)
        vc16 = jnp.cos(th_b).astype(jnp.bfloat16)
        vs16 = jnp.sin(th_b).astype(jnp.bfloat16)
        p = (cr * uc - ci * us).astype(jnp.bfloat16)
        q = (-(cr * us + ci * uc)).astype(jnp.bfloat16)
        out3 = (out3 + vc16[:, None, :] * p[None, :, :]
                + vs16[:, None, :] * q[None, :, :])

    out_ref[0] = out3.reshape(N, DT).astype(jnp.float32)


@jax.jit
def kernel(x):
    b, n, d = x.shape
    grid = (b, d // DT)
    const_spec = lambda a: pl.BlockSpec(a.shape, lambda i, j: (0, 0))
    return pl.pallas_call(
        _fa_kernel,
        grid=grid,
        in_specs=[pl.BlockSpec((1, N, DT), lambda i, j: (i, 0, j))]
                 + [const_spec(c) for c in _CONSTS],
        out_specs=pl.BlockSpec((1, N, DT), lambda i, j: (i, 0, j)),
        out_shape=jax.ShapeDtypeStruct((b, n, d), jnp.float32),
    )(x, *_CONSTS)


# speedup vs baseline: 1.0333x; 1.0333x over previous
import jax
import jax.numpy as jnp
import numpy as np
from jax.experimental import pallas as pl

N = 8192
N1 = 64
N2 = 128
KL1 = 72
GA = 65
GB = 64
KHA = 33
KHB = 32
F_MAX = N // 2
TOPK = 4
DT = 128


def _dft_consts():
    kl = np.arange(KL1)[:, None]
    n2 = np.arange(N2)[None, :]
    ph = -2.0 * np.pi * ((kl * n2) % N2) / N2
    d1r = np.cos(ph).astype(np.float32)
    d1i = np.sin(ph).astype(np.float32)
    d1r[GA:] = 0.0
    d1i[GA:] = 0.0

    n1 = np.arange(N1)[None, :]
    pht = -2.0 * np.pi * ((np.arange(KL1)[:, None] * n1) % N) / N
    tr = np.cos(pht).astype(np.float32)
    ti = np.sin(pht).astype(np.float32)
    tr[GA:] = 0.0
    ti[GA:] = 0.0

    kh = np.arange(KHA)[:, None]
    phe = -2.0 * np.pi * ((kh * n1) % N1) / N1
    ear = np.cos(phe).astype(np.float32)
    eai = np.sin(phe).astype(np.float32)
    eas = (ear + eai).astype(np.float32)

    khb = np.arange(KHB)[:, None] + 1
    phb = -2.0 * np.pi * ((khb * n1) % N1) / N1
    ebr = np.cos(phb).astype(np.float32)
    ebi = np.sin(phb).astype(np.float32)
    ebs = (ebr + ebi).astype(np.float32)
    return d1r, d1i, tr, ti, ear, eai, eas, ebr, ebi, ebs


_CONSTS = _dft_consts()


def _fold(v, reducer, groups):
    return reducer(v.reshape(groups, DT), axis=0, keepdims=True)


def _fa_kernel(x_ref, d1r_ref, d1i_ref, tr_ref, ti_ref, ear_ref, eai_ref,
               eas_ref, ebr_ref, ebi_ref, ebs_ref, out_ref):
    xb = x_ref[0]
    a2f = xb.reshape(N2, N1 * DT)
    hi_p = jax.lax.Precision.HIGHEST
    gr = jnp.dot(d1r_ref[...], a2f, precision=hi_p)
    gi = jnp.dot(d1i_ref[...], a2f, precision=hi_p)
    gr3 = gr.reshape(KL1, N1, DT)
    gi3 = gi.reshape(KL1, N1, DT)
    tr3 = tr_ref[...][:, :, None]
    ti3 = ti_ref[...][:, :, None]
    hr = gr3 * tr3 - gi3 * ti3
    hi = gr3 * ti3 + gi3 * tr3
    hpr = jnp.transpose(hr[:GA], (1, 0, 2)).reshape(N1, GA * DT)
    hpi = jnp.transpose(hi[:GA], (1, 0, 2)).reshape(N1, GA * DT)
    hps = hpr + hpi
    t1 = jnp.dot(ear_ref[...], hpr, precision=hi_p)
    t2 = jnp.dot(eai_ref[...], hpi, precision=hi_p)
    t3 = jnp.dot(eas_ref[...], hps, precision=hi_p)
    x2ar = t1 - t2
    x2ai = t3 - t1 - t2
    hprS = hpr[:, :GB * DT]
    hpiS = hpi[:, :GB * DT]
    hpd = hprS - hpiS
    u1 = jnp.dot(ebr_ref[...], hprS, precision=hi_p)
    u2 = jnp.dot(ebi_ref[...], hpiS, precision=hi_p)
    u3 = jnp.dot(ebs_ref[...], hpd, precision=hi_p)
    x2br = u1 + u2
    x2bi = u3 - u1 + u2

    big_i = jnp.int32(2**30)
    kh_a = jax.lax.broadcasted_iota(jnp.int32, (KHA, GA * DT), 0)
    ln_a = jax.lax.broadcasted_iota(jnp.int32, (KHA, GA * DT), 1)
    kidx_a = kh_a * N2 + jax.lax.shift_right_logical(ln_a, 7)
    amp_a = x2ar * x2ar + x2ai * x2ai
    amp_a = jnp.where(kidx_a <= F_MAX, amp_a, -1.0)

    kh_b = jax.lax.broadcasted_iota(jnp.int32, (KHB, GB * DT), 0)
    ln_b = jax.lax.broadcasted_iota(jnp.int32, (KHB, GB * DT), 1)
    grp_b = jax.lax.shift_right_logical(ln_b, 7)
    dup_b = grp_b == 0
    kidx_b = jnp.where(dup_b, big_i, (kh_b + 1) * N2 - grp_b)
    amp_b = jnp.where(dup_b, -1.0, x2br * x2br + x2bi * x2bi)

    out3 = jnp.zeros((N2, N1, DT), jnp.bfloat16)
    a_io = jax.lax.broadcasted_iota(jnp.int32, (N1, DT), 0)
    b_io = jax.lax.broadcasted_iota(jnp.int32, (N2, DT), 0)
    wka = amp_a
    wkb = amp_b
    inv_n = np.float32(1.0 / N)
    for _ in range(TOPK):
        m = jnp.maximum(_fold(jnp.max(wka, axis=0, keepdims=True), jnp.max, GA),
                        _fold(jnp.max(wkb, axis=0, keepdims=True), jnp.max, GB))
        m_a = jnp.tile(m, (1, GA))
        m_b = jnp.tile(m, (1, GB))
        cand_a = jnp.where(wka == m_a, kidx_a, big_i)
        cand_b = jnp.where(wkb == m_b, kidx_b, big_i)
        f_sel = jnp.minimum(
            _fold(jnp.min(cand_a, axis=0, keepdims=True), jnp.min, GA),
            _fold(jnp.min(cand_b, axis=0, keepdims=True), jnp.min, GB))
        f_a = jnp.tile(f_sel, (1, GA))
        f_b = jnp.tile(f_sel, (1, GB))
        oh_a = kidx_a == f_a
        oh_b = kidx_b == f_b
        wr = (_fold(jnp.sum(jnp.where(oh_a, x2ar, 0.0), axis=0, keepdims=True),
                    jnp.sum, GA)
              + _fold(jnp.sum(jnp.where(oh_b, x2br, 0.0), axis=0, keepdims=True),
                      jnp.sum, GB))
        wi = (_fold(jnp.sum(jnp.where(oh_a, x2ai, 0.0), axis=0, keepdims=True),
                    jnp.sum, GA)
              + _fold(jnp.sum(jnp.where(oh_b, x2bi, 0.0), axis=0, keepdims=True),
                      jnp.sum, GB))
        wka = jnp.where(oh_a, -2.0, wka)
        wkb = jnp.where(oh_b, -2.0, wkb)
        scale = jnp.where((f_sel == 0) | (f_sel == F_MAX), inv_n,
                          np.float32(2.0 / N))
        cr = wr * scale
        ci = wi * scale
        fa = (a_io * f_sel) & (N - 1)
        th_a = fa.astype(jnp.float32) * np.float32(2.0 * np.pi / N)
        uc = jnp.cos(th_a)
        us = jnp.sin(th_a)
        fb = (b_io * f_sel) & (N2 - 1)
        th_b = fb.astype(jnp.float32) * np.float32(2.0 * np.pi / N2)
        vc16 = jnp.cos(th_b).astype(jnp.bfloat16)
        vs16 = jnp.sin(th_b).astype(jnp.bfloat16)
        p = (cr * uc - ci * us).astype(jnp.bfloat16)
        q = (-(cr * us + ci * uc)).astype(jnp.bfloat16)
        out3 = (out3 + vc16[:, None, :] * p[None, :, :]
                + vs16[:, None, :] * q[None, :, :])

    out_ref[0] = out3.reshape(N, DT).astype(jnp.float32)


@jax.jit
def kernel(x):
    b, n, d = x.shape
    grid = (b, d // DT)
    const_spec = lambda a: pl.BlockSpec(a.shape, lambda i, j: (0, 0))
    return pl.pallas_call(
        _fa_kernel,
        grid=grid,
        in_specs=[pl.BlockSpec((1, N, DT), lambda i, j: (i, 0, j))]
                 + [const_spec(c) for c in _CONSTS],
        out_specs=pl.BlockSpec((1, N, DT), lambda i, j: (i, 0, j)),
        out_shape=jax.ShapeDtypeStruct((b, n, d), jnp.float32),
    )(x, *_CONSTS)


# speedup vs baseline: 1.0660x; 1.0316x over previous
import jax
import jax.numpy as jnp
import numpy as np
from jax.experimental import pallas as pl

N = 8192
N1 = 64
N2 = 128
KL1 = 72
GA = 65
GB = 64
KHA = 33
KHB = 32
F_MAX = N // 2
TOPK = 4
DT = 128


def _dft_consts():
    kl = np.arange(KL1)[:, None]
    n2 = np.arange(N2)[None, :]
    ph = -2.0 * np.pi * ((kl * n2) % N2) / N2
    d1r = np.cos(ph).astype(np.float32)
    d1i = np.sin(ph).astype(np.float32)
    d1r[GA:] = 0.0
    d1i[GA:] = 0.0

    n1 = np.arange(N1)[None, :]
    pht = -2.0 * np.pi * ((np.arange(KL1)[:, None] * n1) % N) / N
    tr = np.cos(pht).astype(np.float32)
    ti = np.sin(pht).astype(np.float32)
    tr[GA:] = 0.0
    ti[GA:] = 0.0

    kh = np.arange(KHA)[:, None]
    phe = -2.0 * np.pi * ((kh * n1) % N1) / N1
    ear = np.cos(phe).astype(np.float32)
    eai = np.sin(phe).astype(np.float32)
    eas = (ear + eai).astype(np.float32)

    khb = np.arange(KHB)[:, None] + 1
    phb = -2.0 * np.pi * ((khb * n1) % N1) / N1
    ebr = np.cos(phb).astype(np.float32)
    ebi = np.sin(phb).astype(np.float32)
    ebs = (ebr + ebi).astype(np.float32)

    def mstack(a, b):
        m = np.zeros((KL1, N1), np.float32)
        m[:KHA] = a
        m[KHB + 8:] = b
        return m

    e1 = mstack(ear, ebr)
    e2 = mstack(eai, ebi)
    return d1r, d1i, tr, ti, e1, e2, eas, ebs


_CONSTS = _dft_consts()


def _fold(v, reducer, groups):
    return reducer(v.reshape(groups, DT), axis=0, keepdims=True)


def _fa_kernel(x_ref, d1r_ref, d1i_ref, tr_ref, ti_ref, e1_ref, e2_ref,
               eas_ref, ebs_ref, out_ref):
    xb = x_ref[0]
    a2f = xb.reshape(N2, N1 * DT)
    hi_p = jax.lax.Precision.HIGHEST
    gr = jnp.dot(d1r_ref[...], a2f, precision=hi_p)
    gi = jnp.dot(d1i_ref[...], a2f, precision=hi_p)
    gr3 = gr.reshape(KL1, N1, DT)
    gi3 = gi.reshape(KL1, N1, DT)
    tr3 = tr_ref[...][:, :, None]
    ti3 = ti_ref[...][:, :, None]
    hr = gr3 * tr3 - gi3 * ti3
    hi = gr3 * ti3 + gi3 * tr3
    hpr = jnp.transpose(hr[:GA], (1, 0, 2)).reshape(N1, GA * DT)
    hpi = jnp.transpose(hi[:GA], (1, 0, 2)).reshape(N1, GA * DT)
    hps = hpr + hpi
    hpd = hpr[:, :GB * DT] - hpi[:, :GB * DT]
    r1 = jnp.dot(e1_ref[...], hpr, precision=hi_p)
    r2 = jnp.dot(e2_ref[...], hpi, precision=hi_p)
    t3 = jnp.dot(eas_ref[...], hps, precision=hi_p)
    u3 = jnp.dot(ebs_ref[...], hpd, precision=hi_p)
    t1 = r1[:KHA]
    t2 = r2[:KHA]
    u1 = r1[KHB + 8:, :GB * DT]
    u2 = r2[KHB + 8:, :GB * DT]
    x2ar = t1 - t2
    x2ai = t3 - t1 - t2
    x2br = u1 + u2
    x2bi = u3 - u1 + u2

    big_i = jnp.int32(2**30)
    kh_a = jax.lax.broadcasted_iota(jnp.int32, (KHA, GA * DT), 0)
    ln_a = jax.lax.broadcasted_iota(jnp.int32, (KHA, GA * DT), 1)
    kidx_a = kh_a * N2 + jax.lax.shift_right_logical(ln_a, 7)
    amp_a = x2ar * x2ar + x2ai * x2ai
    amp_a = jnp.where(kidx_a <= F_MAX, amp_a, -1.0)

    kh_b = jax.lax.broadcasted_iota(jnp.int32, (KHB, GB * DT), 0)
    ln_b = jax.lax.broadcasted_iota(jnp.int32, (KHB, GB * DT), 1)
    grp_b = jax.lax.shift_right_logical(ln_b, 7)
    dup_b = grp_b == 0
    kidx_b = jnp.where(dup_b, big_i, (kh_b + 1) * N2 - grp_b)
    amp_b = jnp.where(dup_b, -1.0, x2br * x2br + x2bi * x2bi)

    out3 = jnp.zeros((N2, N1, DT), jnp.bfloat16)
    a_io = jax.lax.broadcasted_iota(jnp.int32, (N1, DT), 0)
    b_io = jax.lax.broadcasted_iota(jnp.int32, (N2, DT), 0)
    wka = amp_a
    wkb = amp_b
    inv_n = np.float32(1.0 / N)
    for _ in range(TOPK):
        m = jnp.maximum(_fold(jnp.max(wka, axis=0, keepdims=True), jnp.max, GA),
                        _fold(jnp.max(wkb, axis=0, keepdims=True), jnp.max, GB))
        m_a = jnp.tile(m, (1, GA))
        m_b = jnp.tile(m, (1, GB))
        cand_a = jnp.where(wka == m_a, kidx_a, big_i)
        cand_b = jnp.where(wkb == m_b, kidx_b, big_i)
        f_sel = jnp.minimum(
            _fold(jnp.min(cand_a, axis=0, keepdims=True), jnp.min, GA),
            _fold(jnp.min(cand_b, axis=0, keepdims=True), jnp.min, GB))
        f_a = jnp.tile(f_sel, (1, GA))
        f_b = jnp.tile(f_sel, (1, GB))
        oh_a = kidx_a == f_a
        oh_b = kidx_b == f_b
        wr = (_fold(jnp.sum(jnp.where(oh_a, x2ar, 0.0), axis=0, keepdims=True),
                    jnp.sum, GA)
              + _fold(jnp.sum(jnp.where(oh_b, x2br, 0.0), axis=0, keepdims=True),
                      jnp.sum, GB))
        wi = (_fold(jnp.sum(jnp.where(oh_a, x2ai, 0.0), axis=0, keepdims=True),
                    jnp.sum, GA)
              + _fold(jnp.sum(jnp.where(oh_b, x2bi, 0.0), axis=0, keepdims=True),
                      jnp.sum, GB))
        wka = jnp.where(oh_a, -2.0, wka)
        wkb = jnp.where(oh_b, -2.0, wkb)
        scale = jnp.where((f_sel == 0) | (f_sel == F_MAX), inv_n,
                          np.float32(2.0 / N))
        cr = wr * scale
        ci = wi * scale
        fa = (a_io * f_sel) & (N - 1)
        th_a = fa.astype(jnp.float32) * np.float32(2.0 * np.pi / N)
        uc = jnp.cos(th_a)
        us = jnp.sin(th_a)
        fb = (b_io * f_sel) & (N2 - 1)
        th_b = fb.astype(jnp.float32) * np.float32(2.0 * np.pi / N2)
        vc16 = jnp.cos(th_b).astype(jnp.bfloat16)
        vs16 = jnp.sin(th_b).astype(jnp.bfloat16)
        p = (cr * uc - ci * us).astype(jnp.bfloat16)
        q = (-(cr * us + ci * uc)).astype(jnp.bfloat16)
        out3 = (out3 + vc16[:, None, :] * p[None, :, :]
                + vs16[:, None, :] * q[None, :, :])

    out_ref[0] = out3.reshape(N, DT).astype(jnp.float32)


@jax.jit
def kernel(x):
    b, n, d = x.shape
    grid = (b, d // DT)
    const_spec = lambda a: pl.BlockSpec(a.shape, lambda i, j: (0, 0))
    return pl.pallas_call(
        _fa_kernel,
        grid=grid,
        in_specs=[pl.BlockSpec((1, N, DT), lambda i, j: (i, 0, j))]
                 + [const_spec(c) for c in _CONSTS],
        out_specs=pl.BlockSpec((1, N, DT), lambda i, j: (i, 0, j)),
        out_shape=jax.ShapeDtypeStruct((b, n, d), jnp.float32),
    )(x, *_CONSTS)


# speedup vs baseline: 1.0769x; 1.0102x over previous
import jax
import jax.numpy as jnp
import numpy as np
from jax.experimental import pallas as pl

N = 8192
N1 = 64
N2 = 128
KL1 = 72
GA = 65
GB = 64
KHA = 33
KHB = 32
F_MAX = N // 2
TOPK = 4
DT = 128


def _dft_consts():
    kl = np.arange(KL1)[:, None]
    n2 = np.arange(N2)[None, :]
    ph = -2.0 * np.pi * ((kl * n2) % N2) / N2
    d1r = np.cos(ph).astype(np.float32)
    d1i = np.sin(ph).astype(np.float32)
    d1r[GA:] = 0.0
    d1i[GA:] = 0.0

    n1 = np.arange(N1)[None, :]
    pht = -2.0 * np.pi * ((np.arange(KL1)[:, None] * n1) % N) / N
    tr = np.cos(pht).astype(np.float32)
    ti = np.sin(pht).astype(np.float32)
    tr[GA:] = 0.0
    ti[GA:] = 0.0

    kh = np.arange(KHA)[:, None]
    phe = -2.0 * np.pi * ((kh * n1) % N1) / N1
    ear = np.cos(phe).astype(np.float32)
    eai = np.sin(phe).astype(np.float32)
    eas = (ear + eai).astype(np.float32)

    khb = np.arange(KHB)[:, None] + 1
    phb = -2.0 * np.pi * ((khb * n1) % N1) / N1
    ebr = np.cos(phb).astype(np.float32)
    ebi = np.sin(phb).astype(np.float32)
    ebs = (ebr + ebi).astype(np.float32)

    def mstack(a, b):
        m = np.zeros((KL1, N1), np.float32)
        m[:KHA] = a
        m[KHB + 8:] = b
        return m

    e1 = mstack(ear, ebr)
    e2 = mstack(eai, ebi)
    return d1r, d1i, tr, ti, e1, e2, eas, ebs


_CONSTS = _dft_consts()


def _fold(v, reducer, groups):
    return reducer(v.reshape(groups, DT), axis=0, keepdims=True)


def _fa_kernel(x_ref, d1r_ref, d1i_ref, tr_ref, ti_ref, e1_ref, e2_ref,
               eas_ref, ebs_ref, out_ref):
    xb = x_ref[0]
    a2f = xb.reshape(N2, N1 * DT)
    hi_p = jax.lax.Precision.HIGHEST
    gr = jnp.dot(d1r_ref[...], a2f, precision=hi_p)
    gi = jnp.dot(d1i_ref[...], a2f, precision=hi_p)
    gr3 = gr.reshape(KL1, N1, DT)
    gi3 = gi.reshape(KL1, N1, DT)
    tr3 = tr_ref[...][:, :, None]
    ti3 = ti_ref[...][:, :, None]
    hr = gr3 * tr3 - gi3 * ti3
    hi = gr3 * ti3 + gi3 * tr3
    hpr = jnp.transpose(hr[:GA], (1, 0, 2)).reshape(N1, GA * DT)
    hpi = jnp.transpose(hi[:GA], (1, 0, 2)).reshape(N1, GA * DT)
    hps = hpr + hpi
    hpd = hpr[:, :GB * DT] - hpi[:, :GB * DT]
    r1 = jnp.dot(e1_ref[...], hpr, precision=hi_p)
    r2 = jnp.dot(e2_ref[...], hpi, precision=hi_p)
    t3 = jnp.dot(eas_ref[...], hps, precision=hi_p)
    u3 = jnp.dot(ebs_ref[...], hpd, precision=hi_p)
    t1 = r1[:KHA]
    t2 = r2[:KHA]
    u1 = r1[KHB + 8:, :GB * DT]
    u2 = r2[KHB + 8:, :GB * DT]
    x2ar = t1 - t2
    x2ai = t3 - t1 - t2
    x2br = u1 + u2
    x2bi = u3 - u1 + u2

    big_i = jnp.int32(2**30)
    kh_a = jax.lax.broadcasted_iota(jnp.int32, (KHA, GA * DT), 0)
    ln_a = jax.lax.broadcasted_iota(jnp.int32, (KHA, GA * DT), 1)
    kidx_a = kh_a * N2 + jax.lax.shift_right_logical(ln_a, 7)
    amp_a = x2ar * x2ar + x2ai * x2ai
    amp_a = jnp.where(kidx_a <= F_MAX, amp_a, -1.0)

    kh_b = jax.lax.broadcasted_iota(jnp.int32, (KHB, GB * DT), 0)
    ln_b = jax.lax.broadcasted_iota(jnp.int32, (KHB, GB * DT), 1)
    grp_b = jax.lax.shift_right_logical(ln_b, 7)
    dup_b = grp_b == 0
    kidx_b = jnp.where(dup_b, big_i, (kh_b + 1) * N2 - grp_b)
    amp_b = jnp.where(dup_b, -1.0, x2br * x2br + x2bi * x2bi)

    out3 = None
    a_io = jax.lax.broadcasted_iota(jnp.int32, (N1, DT), 0)
    b_io = jax.lax.broadcasted_iota(jnp.int32, (N2, DT), 0)
    wka = amp_a
    wkb = amp_b
    inv_n = np.float32(1.0 / N)
    for _ in range(TOPK):
        m = jnp.maximum(_fold(jnp.max(wka, axis=0, keepdims=True), jnp.max, GA),
                        _fold(jnp.max(wkb, axis=0, keepdims=True), jnp.max, GB))
        m_a = jnp.tile(m, (1, GA))
        m_b = jnp.tile(m, (1, GB))
        cand_a = jnp.where(wka == m_a, kidx_a, big_i)
        cand_b = jnp.where(wkb == m_b, kidx_b, big_i)
        f_sel = jnp.minimum(
            _fold(jnp.min(cand_a, axis=0, keepdims=True), jnp.min, GA),
            _fold(jnp.min(cand_b, axis=0, keepdims=True), jnp.min, GB))
        f_a = jnp.tile(f_sel, (1, GA))
        f_b = jnp.tile(f_sel, (1, GB))
        oh_a = kidx_a == f_a
        oh_b = kidx_b == f_b
        wr = (_fold(jnp.sum(jnp.where(oh_a, x2ar, 0.0), axis=0, keepdims=True),
                    jnp.sum, GA)
              + _fold(jnp.sum(jnp.where(oh_b, x2br, 0.0), axis=0, keepdims=True),
                      jnp.sum, GB))
        wi = (_fold(jnp.sum(jnp.where(oh_a, x2ai, 0.0), axis=0, keepdims=True),
                    jnp.sum, GA)
              + _fold(jnp.sum(jnp.where(oh_b, x2bi, 0.0), axis=0, keepdims=True),
                      jnp.sum, GB))
        wka = jnp.where(oh_a, -2.0, wka)
        wkb = jnp.where(oh_b, -2.0, wkb)
        scale = jnp.where((f_sel == 0) | (f_sel == F_MAX), inv_n,
                          np.float32(2.0 / N))
        cr = wr * scale
        ci = wi * scale
        fa = (a_io * f_sel) & (N - 1)
        th_a = fa.astype(jnp.float32) * np.float32(2.0 * np.pi / N)
        uc = jnp.cos(th_a)
        us = jnp.sin(th_a)
        fb = (b_io * f_sel) & (N2 - 1)
        th_b = fb.astype(jnp.float32) * np.float32(2.0 * np.pi / N2)
        vc16 = jnp.cos(th_b).astype(jnp.bfloat16)
        vs16 = jnp.sin(th_b).astype(jnp.bfloat16)
        p = (cr * uc - ci * us).astype(jnp.bfloat16)
        q = (-(cr * us + ci * uc)).astype(jnp.bfloat16)
        term = (vc16[:, None, :] * p[None, :, :]
                + vs16[:, None, :] * q[None, :, :])
        out3 = term if out3 is None else out3 + term

    out_ref[0] = out3.reshape(N, DT).astype(jnp.float32)


@jax.jit
def kernel(x):
    b, n, d = x.shape
    grid = (b, d // DT)
    const_spec = lambda a: pl.BlockSpec(a.shape, lambda i, j: (0, 0))
    return pl.pallas_call(
        _fa_kernel,
        grid=grid,
        in_specs=[pl.BlockSpec((1, N, DT), lambda i, j: (i, 0, j))]
                 + [const_spec(c) for c in _CONSTS],
        out_specs=pl.BlockSpec((1, N, DT), lambda i, j: (i, 0, j)),
        out_shape=jax.ShapeDtypeStruct((b, n, d), jnp.float32),
    )(x, *_CONSTS)
